# baseline (device time: 44851 ns/iter reference)
import jax
import jax.numpy as jnp
from jax import lax
from jax.experimental import pallas as pl
from jax.experimental.pallas import tpu as pltpu

N_DEV = 32
A = 4
B = 8


def kernel(x, w_mat):
    k_full, blk = x.shape
    _, n = w_mat.shape
    slab_rows = B * blk

    def body(x_ref, w_ref, out_ref, slab_ref, gather_ref,
             p1_send, p1_recv, p2_send, p2_recv):
        me = lax.axis_index("i")
        a = me // B
        b = me % B

        slab_ref[a] = x_ref[pl.ds(a * slab_rows, slab_rows), :]

        for da in range(1, A):
            a_d = (a + da) % A
            rdma = pltpu.make_async_remote_copy(
                src_ref=x_ref.at[pl.ds(a_d * slab_rows, slab_rows), :],
                dst_ref=slab_ref.at[a],
                send_sem=p1_send.at[da],
                recv_sem=p1_recv.at[da],
                device_id=(a_d * B + b,),
                device_id_type=pl.DeviceIdType.MESH,
            )
            rdma.start()

        for da in range(1, A):
            src_a = (a - da) % A
            rdma = pltpu.make_async_remote_copy(
                src_ref=x_ref.at[pl.ds(0, slab_rows), :],
                dst_ref=slab_ref.at[src_a],
                send_sem=p1_send.at[da],
                recv_sem=p1_recv.at[da],
                device_id=(src_a * B + b,),
                device_id_type=pl.DeviceIdType.MESH,
            )
            rdma.wait_recv()

        gather_ref[b] = slab_ref[:, pl.ds(b * blk, blk), :]

        for db in range(1, B):
            b_d = (b + db) % B
            rdma = pltpu.make_async_remote_copy(
                src_ref=slab_ref.at[:, pl.ds(b_d * blk, blk), :],
                dst_ref=gather_ref.at[b],
                send_sem=p2_send.at[db],
                recv_sem=p2_recv.at[db],
                device_id=(a * B + b_d,),
                device_id_type=pl.DeviceIdType.MESH,
            )
            rdma.start()

        for db in range(1, B):
            src_b = (b - db) % B
            rdma = pltpu.make_async_remote_copy(
                src_ref=slab_ref.at[:, pl.ds(0, blk), :],
                dst_ref=gather_ref.at[src_b],
                send_sem=p2_send.at[db],
                recv_sem=p2_recv.at[db],
                device_id=(a * B + src_b,),
                device_id_type=pl.DeviceIdType.MESH,
            )
            rdma.wait_recv()

        for da in range(1, A):
            a_d = (a + da) % A
            rdma = pltpu.make_async_remote_copy(
                src_ref=x_ref.at[pl.ds(a_d * slab_rows, slab_rows), :],
                dst_ref=slab_ref.at[a],
                send_sem=p1_send.at[da],
                recv_sem=p1_recv.at[da],
                device_id=(a_d * B + b,),
                device_id_type=pl.DeviceIdType.MESH,
            )
            rdma.wait_send()
        for db in range(1, B):
            b_d = (b + db) % B
            rdma = pltpu.make_async_remote_copy(
                src_ref=slab_ref.at[:, pl.ds(b_d * blk, blk), :],
                dst_ref=gather_ref.at[b],
                send_sem=p2_send.at[db],
                recv_sem=p2_recv.at[db],
                device_id=(a * B + b_d,),
                device_id_type=pl.DeviceIdType.MESH,
            )
            rdma.wait_send()

        g = gather_ref[:, :, :, :]
        assembled = jnp.transpose(g, (2, 1, 0, 3)).reshape(blk, k_full)
        out_ref[:, :] = jnp.dot(
            assembled, w_ref[:, :], preferred_element_type=jnp.float32
        )

    return pl.pallas_call(
        body,
        out_shape=jax.ShapeDtypeStruct((blk, n), jnp.float32),
        in_specs=[
            pl.BlockSpec(memory_space=pltpu.VMEM),
            pl.BlockSpec(memory_space=pltpu.VMEM),
        ],
        out_specs=pl.BlockSpec(memory_space=pltpu.VMEM),
        scratch_shapes=[
            pltpu.VMEM((A, slab_rows, blk), x.dtype),
            pltpu.VMEM((B, A, blk, blk), x.dtype),
            pltpu.SemaphoreType.DMA((A,)),
            pltpu.SemaphoreType.DMA((A,)),
            pltpu.SemaphoreType.DMA((B,)),
            pltpu.SemaphoreType.DMA((B,)),
        ],
    )(x, w_mat)


# device time: 30072 ns/iter; 1.4915x vs baseline; 1.4915x over previous
import jax
import jax.numpy as jnp
from jax import lax
from jax.experimental import pallas as pl
from jax.experimental.pallas import tpu as pltpu

N_DEV = 32


def kernel(x, w_mat):
    k_full, blk = x.shape
    _, n = w_mat.shape
    rows2 = blk // 2
    x_dense = x.reshape(k_full // 2, 2 * blk)

    def body(x_ref, w_ref, out_ref, gather_ref, send_sems, recv_sems):
        me = lax.axis_index("i")

        gather_ref[me] = x_ref[pl.ds(me * rows2, rows2), :]

        for off in range(1, N_DEV):
            dst = (me + off) % N_DEV
            rdma = pltpu.make_async_remote_copy(
                src_ref=x_ref.at[pl.ds(dst * rows2, rows2), :],
                dst_ref=gather_ref.at[me],
                send_sem=send_sems.at[off],
                recv_sem=recv_sems.at[off],
                device_id=(dst,),
                device_id_type=pl.DeviceIdType.MESH,
            )
            rdma.start()

        for off in range(1, N_DEV):
            src_dev = (me - off) % N_DEV
            rdma = pltpu.make_async_remote_copy(
                src_ref=x_ref.at[pl.ds(0, rows2), :],
                dst_ref=gather_ref.at[src_dev],
                send_sem=send_sems.at[off],
                recv_sem=recv_sems.at[off],
                device_id=(src_dev,),
                device_id_type=pl.DeviceIdType.MESH,
            )
            rdma.wait_recv()

        for off in range(1, N_DEV):
            dst = (me + off) % N_DEV
            rdma = pltpu.make_async_remote_copy(
                src_ref=x_ref.at[pl.ds(dst * rows2, rows2), :],
                dst_ref=gather_ref.at[me],
                send_sem=send_sems.at[off],
                recv_sem=recv_sems.at[off],
                device_id=(dst,),
                device_id_type=pl.DeviceIdType.MESH,
            )
            rdma.wait_send()

        g4 = gather_ref[:, :, :].reshape(N_DEV, rows2, 2, blk)
        assembled = jnp.transpose(g4, (1, 2, 0, 3)).reshape(2 * rows2, N_DEV * blk)
        out_ref[:, :] = jnp.dot(
            assembled, w_ref[:, :], preferred_element_type=jnp.float32
        )

    return pl.pallas_call(
        body,
        out_shape=jax.ShapeDtypeStruct((blk, n), jnp.float32),
        in_specs=[
            pl.BlockSpec(memory_space=pltpu.VMEM),
            pl.BlockSpec(memory_space=pltpu.VMEM),
        ],
        out_specs=pl.BlockSpec(memory_space=pltpu.VMEM),
        scratch_shapes=[
            pltpu.VMEM((N_DEV, rows2, 2 * blk), x.dtype),
            pltpu.SemaphoreType.DMA((N_DEV,)),
            pltpu.SemaphoreType.DMA((N_DEV,)),
        ],
    )(x_dense, w_mat)
